# Initial kernel scaffold; baseline (speedup 1.0000x reference)
import jax
import jax.numpy as jnp
from jax.experimental import pallas as pl


def kernel(feats, edge_index, dest, batch, W, b):
    raise NotImplementedError("write your pallas kernel here")


# speedup vs baseline: 40.4060x; 40.4060x over previous
import functools

import jax
import jax.numpy as jnp
from jax import lax
from jax.experimental import pallas as pl
from jax.experimental.pallas import tpu as pltpu
from jax.experimental.pallas import tpu_sc as plsc

N_NODES = 10000
N_EDGES = 320000
N_FEATS = 128
N_GRAPHS = 200

N_TILES = 16
NODES_PAD = 10240
NODE_CHUNK = NODES_PAD // N_TILES
E_PER_TILE = N_EDGES // N_TILES
EVREGS = E_PER_TILE // 16
GRAPHS_PAD = 208


def _util_body(f_ref, w_ref, b_ref, o_ref):
    prod = jax.lax.dot_general(
        f_ref[...], w_ref[...],
        dimension_numbers=(((1,), (0,)), ((), ())),
        preferred_element_type=jnp.float32,
    )
    o_ref[...] = prod.reshape(8, 128) + b_ref[0, 0]


def _util_tc(feats, W, b):
    rows = N_EDGES // 128
    grid = pl.cdiv(rows, 8)
    out = pl.pallas_call(
        _util_body,
        grid=(grid,),
        in_specs=[
            pl.BlockSpec((1024, N_FEATS), lambda i: (i, 0)),
            pl.BlockSpec((N_FEATS, 1), lambda i: (0, 0)),
            pl.BlockSpec(memory_space=pltpu.SMEM),
        ],
        out_specs=pl.BlockSpec((8, 128), lambda i: (i, 0)),
        out_shape=jax.ShapeDtypeStruct((rows, 128), jnp.float32),
    )(feats, W.reshape(N_FEATS, 1), b.reshape(1, 1))
    return out.reshape(N_EDGES)


def _sc_body(src_hbm, dst_hbm, util_hbm, dest_hbm, batch_hbm,
             out_hbm,
             src_v, dst_v, util_v,
             value_v, acc_v,
             batch_v, hist_v, dest_v,
             merged_v, gathm_v,
             kscr_v,
             accs_sh, value_sh):
    sid = lax.axis_index("s")
    iota = lax.iota(jnp.int32, 16)
    zeros_i = jnp.zeros((16,), jnp.int32)
    ones_i = jnp.ones((16,), jnp.int32)
    ninf = jnp.full((16,), -jnp.inf, jnp.float32)

    ebase = sid * E_PER_TILE
    pltpu.sync_copy(src_hbm.at[pl.ds(ebase, E_PER_TILE)], src_v)
    pltpu.sync_copy(dst_hbm.at[pl.ds(ebase, E_PER_TILE)], dst_v)
    pltpu.sync_copy(util_hbm.at[pl.ds(ebase, E_PER_TILE)], util_v)
    pltpu.sync_copy(dest_hbm, dest_v)
    pltpu.sync_copy(batch_hbm, batch_v)

    def zero_hist(i, c):
        hist_v[pl.ds(i * 16, 16)] = zeros_i
        return c
    lax.fori_loop(0, GRAPHS_PAD // 16, zero_hist, 0)

    def hist_step(i, c):
        b = batch_v[pl.ds(i * 16, 16)]
        plsc.addupdate_scatter(hist_v, [b], ones_i)
        return c
    lax.fori_loop(0, N_NODES // 16, hist_step, 0)

    def hist_max(i, m):
        return jnp.maximum(m, hist_v[pl.ds(i * 16, 16)])
    n_steps = jnp.max(lax.fori_loop(0, GRAPHS_PAD // 16, hist_max, zeros_i)) - 1

    def init_value(i, c):
        value_v[pl.ds(i * 16, 16)] = ninf
        return c
    lax.fori_loop(0, NODES_PAD // 16, init_value, 0)

    def set_dest(i, c):
        d = dest_v[pl.ds(i * 16, 16)]
        plsc.store_scatter(value_v, [d], jnp.zeros((16,), jnp.float32))
        return c
    lax.fori_loop(0, GRAPHS_PAD // 16, set_dest, 0)

    def bf_step(_, c):
        def init_acc(i, cc):
            acc_v[pl.ds(i * 16, 16)] = ninf
            return cc
        lax.fori_loop(0, NODES_PAD // 16, init_acc, 0)

        def edge_body(j, cc):
            e = j * 16
            s = src_v[pl.ds(e, 16)]
            d = dst_v[pl.ds(e, 16)]
            u = util_v[pl.ds(e, 16)]
            m = plsc.load_gather(value_v, [d]) + u
            ks, ms = plsc.sort_key_val(s, m)
            kscr_v[...] = ks
            def lift(dd, mm):
                idx = jnp.minimum(iota + dd, 15)
                kshift = plsc.load_gather(kscr_v, [idx])
                mshift = jax.lax.gather(
                    mm, idx.reshape(16, 1),
                    jax.lax.GatherDimensionNumbers(
                        offset_dims=(), collapsed_slice_dims=(0,),
                        start_index_map=(0,)),
                    slice_sizes=(1,),
                    mode=jax.lax.GatherScatterMode.PROMISE_IN_BOUNDS,
                )
                return jnp.where(kshift == ks, jnp.maximum(mm, mshift), mm)
            mm = lift(1, ms)
            mm = lift(2, mm)
            mm = lift(4, mm)
            mm = lift(8, mm)
            kprev = plsc.load_gather(kscr_v, [jnp.maximum(iota - 1, 0)])
            run_start = (iota == 0) | (kprev != ks)
            old = plsc.load_gather(acc_v, [ks])
            plsc.store_scatter(acc_v, [ks], jnp.maximum(old, mm),
                               mask=run_start)
            return cc
        lax.fori_loop(0, EVREGS, edge_body, 0)

        pltpu.sync_copy(acc_v, accs_sh.at[sid])
        plsc.subcore_barrier()
        for t in range(N_TILES):
            pltpu.sync_copy(accs_sh.at[t, pl.ds(sid * NODE_CHUNK, NODE_CHUNK)],
                            gathm_v.at[t])

        def merge_body(ci, cc):
            col = ci * 16
            m = gathm_v[0, pl.ds(col, 16)]
            for t in range(1, N_TILES):
                m = jnp.maximum(m, gathm_v[t, pl.ds(col, 16)])
            merged_v[pl.ds(col, 16)] = m
            return cc
        lax.fori_loop(0, NODE_CHUNK // 16, merge_body, 0)

        pltpu.sync_copy(merged_v, value_sh.at[pl.ds(sid * NODE_CHUNK,
                                                    NODE_CHUNK)])
        plsc.subcore_barrier()
        pltpu.sync_copy(value_sh, value_v)
        return c

    lax.fori_loop(0, n_steps, bf_step, 0)

    pltpu.sync_copy(value_v.at[pl.ds(sid * NODE_CHUNK, NODE_CHUNK)],
                    out_hbm.at[pl.ds(sid * NODE_CHUNK, NODE_CHUNK)])


def _sc_bellman_ford(src, dst, util, dest_p, batch):
    mesh = plsc.VectorSubcoreMesh(core_axis_name="c", subcore_axis_name="s",
                                  num_cores=1)
    fn = pl.kernel(
        _sc_body,
        out_type=jax.ShapeDtypeStruct((NODES_PAD,), jnp.float32),
        mesh=mesh,
        compiler_params=pltpu.CompilerParams(needs_layout_passes=False),
        scratch_types=[
            pltpu.VMEM((E_PER_TILE,), jnp.int32),
            pltpu.VMEM((E_PER_TILE,), jnp.int32),
            pltpu.VMEM((E_PER_TILE,), jnp.float32),
            pltpu.VMEM((NODES_PAD,), jnp.float32),
            pltpu.VMEM((NODES_PAD,), jnp.float32),
            pltpu.VMEM((N_NODES,), jnp.int32),
            pltpu.VMEM((GRAPHS_PAD,), jnp.int32),
            pltpu.VMEM((GRAPHS_PAD,), jnp.int32),
            pltpu.VMEM((NODE_CHUNK,), jnp.float32),
            pltpu.VMEM((N_TILES, NODE_CHUNK), jnp.float32),
            pltpu.VMEM((16,), jnp.int32),
            pltpu.VMEM_SHARED((N_TILES, NODES_PAD), jnp.float32),
            pltpu.VMEM_SHARED((NODES_PAD,), jnp.float32),
        ],
    )
    return fn(src, dst, util, dest_p, batch)


def kernel(feats, edge_index, dest, batch, W, b):
    util = _util_tc(feats, W, b)
    src = edge_index[0]
    dst = edge_index[1]
    dest_p = jnp.concatenate(
        [dest, jnp.broadcast_to(dest[:1], (GRAPHS_PAD - N_GRAPHS,))])
    value_p = _sc_bellman_ford(src, dst, util, dest_p, batch)
    return value_p[:N_NODES].reshape(N_NODES, 1), util.reshape(N_EDGES, 1)
